# baseline (speedup 1.0000x reference)
import math

import jax
import jax.numpy as jnp
from jax import lax
from jax.experimental import pallas as pl
from jax.experimental.pallas import tpu as pltpu
from jax.experimental.pallas import tpu_sc as plsc

NUM_EMB = 1000000
DIM = 64
BATCH = 4096
SEQ = 200

NC = 2
NS = 16
NW = NC * NS
BW = BATCH // NW

NBUF = 4
L = 16


def _pe_table():
    position = jnp.arange(0.0, SEQ)[:, None]
    div_term = jnp.exp(
        jnp.arange(0.0, DIM, 2) * -(math.log(10000.0) / DIM))
    tmp = position * div_term
    pe = jnp.zeros((SEQ, DIM), dtype=jnp.float32)
    pe = pe.at[:, 0::2].set(jnp.sin(tmp))
    pe = pe.at[:, 1::2].set(jnp.cos(tmp))
    return pe


def _body(table_hbm, idxt_hbm, pe_hbm, out_hbm, idx_v, pe_v, gbuf, tbuf,
          *sems):
    gsem = sems[:NBUF]
    ssem = sems[NBUF:]
    wid = lax.axis_index("c") * NS + lax.axis_index("s")
    base = wid * BW

    pltpu.sync_copy(idxt_hbm.at[:, pl.ds(base, BW)], idx_v)
    pltpu.sync_copy(pe_hbm, pe_v)

    def issue_gather(s, p):
        pltpu.async_copy(table_hbm.at[idx_v.at[s]], gbuf.at[p], gsem[p])

    def wait_gather(s, p):
        pltpu.make_async_copy(
            table_hbm.at[idx_v.at[s]], gbuf.at[p], gsem[p]).wait()

    def issue_store(s, p):
        pltpu.async_copy(tbuf.at[p], out_hbm.at[s, :, pl.ds(base, BW)],
                         ssem[p])

    def wait_store(s, p):
        pltpu.make_async_copy(
            tbuf.at[p], out_hbm.at[s, :, pl.ds(base, BW)], ssem[p]).wait()

    for p in range(NBUF):
        issue_gather(p, p)

    lanes = lax.iota(jnp.int32, L)

    @pl.loop(0, SEQ // NBUF)
    def _sstep(ss):
        for p in range(NBUF):
            s = ss * NBUF + p

            @pl.when(s >= NBUF)
            def _():
                wait_store(s - NBUF, p)

            wait_gather(s, p)

            pe_c = [pe_v[s, pl.ds(c * L, L)] for c in range(DIM // L)]
            d_idx = [lanes + c * L for c in range(DIM // L)]

            @pl.loop(0, BW)
            def _col(b):
                b_idx = jnp.full((L,), 0, jnp.int32) + b
                for c in range(DIM // L):
                    v = gbuf[p, b, pl.ds(c * L, L)] + pe_c[c]
                    plsc.store_scatter(tbuf.at[p], [d_idx[c], b_idx], v)

            issue_store(s, p)

            @pl.when(s + NBUF < SEQ)
            def _():
                issue_gather(s + NBUF, p)

    for p in range(NBUF):
        wait_store(SEQ - NBUF + p, p)


def kernel(inputs, table):
    pe = _pe_table()
    idxt = inputs.astype(jnp.int32).T
    mesh = plsc.VectorSubcoreMesh(core_axis_name="c", subcore_axis_name="s")
    kfn = pl.kernel(
        _body,
        out_type=jax.ShapeDtypeStruct((SEQ, DIM, BATCH), jnp.float32),
        mesh=mesh,
        scratch_types=(
            [pltpu.VMEM((SEQ, BW), jnp.int32),
             pltpu.VMEM((SEQ, DIM), jnp.float32),
             pltpu.VMEM((NBUF, BW, DIM), jnp.float32),
             pltpu.VMEM((NBUF, DIM, BW), jnp.float32)]
            + [pltpu.SemaphoreType.DMA] * (2 * NBUF)),
        compiler_params=pltpu.CompilerParams(
            use_tc_tiling_on_sc=False, needs_layout_passes=False),
    )
    out_t = kfn(table, idxt, pe)
    return jnp.transpose(out_t, (2, 0, 1))
